# baseline (speedup 1.0000x reference)
import functools

import jax
import jax.numpy as jnp
from jax.experimental import pallas as pl
from jax.experimental.pallas import tpu as pltpu

_B, _S, _P, _C, _K = 16, 720, 336, 321, 8
_T = _S + _P
_TP = 1152
_W = 512
_CP = 384
_CBLK = 128
_CC = 16
_CG = 336
_EPS = 1e-5


def _stage_a_body(xt_ref, w_ref, seq_ref, mean_ref, std_ref):
    xb = xt_ref[...]
    mean = jnp.mean(xb, axis=2)
    xc = xb - mean[:, :, None]
    var = jnp.mean(xc * xc, axis=2)
    std = jnp.sqrt(var + _EPS)
    xn = xc / std[:, :, None]
    mean_ref[...] = mean
    std_ref[...] = std
    seq_ref[:, :, :_S] = xn
    xn2 = xn.reshape(_CBLK * _B, _S)
    yh = jax.lax.dot_general(xn2, w_ref[...],
                             (((1,), (1,)), ((), ())),
                             preferred_element_type=jnp.float32)
    seq_ref[:, :, _S:_T] = yh.reshape(_CBLK, _B, _P)
    seq_ref[:, :, _T:] = jnp.zeros((_CBLK, _B, _TP - _T), jnp.float32)


def _stage_b_body(seq_hbm, leaders_ref, starts_ref, deltas_ref, wf_ref,
                  wc_ref, lw_ref, mean_ref, std_ref, y_ref, ss_ref,
                  seq_ref, sem):
    cb = pl.program_id(0)

    @pl.when(cb == 0)
    def _():
        cp = pltpu.make_async_copy(seq_hbm, seq_ref, sem)
        cp.start()
        cp.wait()

    for i in range(_CC):
        c = cb * _CC + i
        yh_c = seq_ref[c, :, _S:_T]
        ss_ref[i, 0] = yh_c

        lw = lw_ref[i]
        lw_max = jnp.max(lw, axis=1, keepdims=True)
        e = jnp.exp(lw - lw_max)
        w = e / jnp.sum(e, axis=1, keepdims=True)

        acc = w[0:1, 0:1] * yh_c
        for k in range(_K):
            ch = leaders_ref[c, k]
            st = starts_ref[c, k]
            d = deltas_ref[c, k]
            base = pl.multiple_of((st // 128) * 128, 128)
            off = st - base
            window = seq_ref[ch, :, pl.ds(base, _W)]
            gc = pltpu.roll(window, _W - off, axis=1)[:, :_P]
            gf = pltpu.roll(window, _W - (off + d), axis=1)[:, :_P]
            out_k = gf * wf_ref[c, k] + gc * wc_ref[c, k]
            ss_ref[i, 1 + k] = out_k
            acc = acc + w[0:1, 1 + k:2 + k] * out_k

        mean_c = mean_ref[i]
        std_c = std_ref[i]
        y_ref[i] = acc * std_c + mean_c


@jax.jit
def kernel(x, leaders, shifts, W, leader_weight):
    sh = jax.nn.sigmoid(shifts) * _S
    sf = jnp.floor(sh)
    sc = jnp.ceil(sh)
    padr = ((0, _CG - _C), (0, 0))
    starts = jnp.pad((_S - sc.astype(jnp.int32)).reshape(_C, _K), padr)
    deltas = jnp.pad((sc - sf).astype(jnp.int32).reshape(_C, _K), padr)
    wf = jnp.pad((sh - sf).reshape(_C, _K), padr)
    wc = jnp.pad((sh + 1.0 - sc).reshape(_C, _K), padr)
    leaders2 = jnp.pad(leaders.reshape(_C, _K), padr)

    x_t = jnp.pad(x.transpose(2, 0, 1), ((0, _CP - _C), (0, 0), (0, 0)))

    grid_a = (_CP // _CBLK,)
    seq_cbt, mean_cb, std_cb = pl.pallas_call(
        _stage_a_body,
        grid=grid_a,
        in_specs=[
            pl.BlockSpec((_CBLK, _B, _S), lambda i: (i, 0, 0)),
            pl.BlockSpec((_P, _S), lambda i: (0, 0)),
        ],
        out_specs=[
            pl.BlockSpec((_CBLK, _B, _TP), lambda i: (i, 0, 0)),
            pl.BlockSpec((_CBLK, _B), lambda i: (i, 0)),
            pl.BlockSpec((_CBLK, _B), lambda i: (i, 0)),
        ],
        out_shape=[
            jax.ShapeDtypeStruct((_CP, _B, _TP), jnp.float32),
            jax.ShapeDtypeStruct((_CP, _B), jnp.float32),
            jax.ShapeDtypeStruct((_CP, _B), jnp.float32),
        ],
    )(x_t, W)

    mean3 = mean_cb.reshape(_CP, _B, 1)
    std3 = std_cb.reshape(_CP, _B, 1)
    lw3 = jnp.pad(leader_weight, ((0, _CG - _C), (0, 0))).reshape(_CG, 1, 1 + _K)

    smem = functools.partial(pl.BlockSpec, memory_space=pltpu.SMEM)
    y_t, ss = pl.pallas_call(
        _stage_b_body,
        grid=(_CG // _CC,),
        in_specs=[
            pl.BlockSpec(memory_space=pltpu.MemorySpace.HBM),
            smem(),
            smem(),
            smem(),
            smem(),
            smem(),
            pl.BlockSpec((_CC, 1, 1 + _K), lambda c: (c, 0, 0)),
            pl.BlockSpec((_CC, _B, 1), lambda c: (c, 0, 0)),
            pl.BlockSpec((_CC, _B, 1), lambda c: (c, 0, 0)),
        ],
        out_specs=[
            pl.BlockSpec((_CC, _B, _P), lambda c: (c, 0, 0)),
            pl.BlockSpec((_CC, 1 + _K, _B, _P), lambda c: (c, 0, 0, 0)),
        ],
        out_shape=[
            jax.ShapeDtypeStruct((_C, _B, _P), jnp.float32),
            jax.ShapeDtypeStruct((_C, 1 + _K, _B, _P), jnp.float32),
        ],
        scratch_shapes=[
            pltpu.VMEM((_CP, _B, _TP), jnp.float32),
            pltpu.SemaphoreType.DMA,
        ],
    )(seq_cbt, leaders2, starts, deltas, wf, wc, lw3, mean3, std3)

    y = y_t.transpose(1, 2, 0)
    seq_shifted = ss.transpose(2, 3, 0, 1)
    return (y, seq_shifted)


# speedup vs baseline: 1.1957x; 1.1957x over previous
import functools

import jax
import jax.numpy as jnp
from jax.experimental import pallas as pl
from jax.experimental.pallas import tpu as pltpu

_B, _S, _P, _C, _K = 16, 720, 336, 321, 8
_T = _S + _P
_TP = 1152
_W = 512
_CP = 384
_CBLK = 128
_CC = 32
_CG = 352
_EPS = 1e-5


def _stage_a_body(xt_ref, w_ref, seq_ref, mean_ref, std_ref):
    xb = xt_ref[...]
    mean = jnp.mean(xb, axis=2)
    xc = xb - mean[:, :, None]
    var = jnp.mean(xc * xc, axis=2)
    std = jnp.sqrt(var + _EPS)
    xn = xc / std[:, :, None]
    mean_ref[...] = mean
    std_ref[...] = std
    seq_ref[:, :, :_S] = xn
    xn2 = xn.reshape(_CBLK * _B, _S)
    yh = jax.lax.dot_general(xn2, w_ref[...],
                             (((1,), (1,)), ((), ())),
                             preferred_element_type=jnp.float32)
    seq_ref[:, :, _S:_T] = yh.reshape(_CBLK, _B, _P)
    seq_ref[:, :, _T:] = jnp.zeros((_CBLK, _B, _TP - _T), jnp.float32)


def _stage_b_body(seq_hbm, leaders_ref, starts_ref, deltas_ref, wf_ref,
                  wc_ref, lw_ref, mean_ref, std_ref, y_ref, ss_ref,
                  seq_ref, sem):
    cb = pl.program_id(0)

    @pl.when(cb == 0)
    def _():
        cp = pltpu.make_async_copy(seq_hbm, seq_ref, sem)
        cp.start()
        cp.wait()

    for i in range(_CC):
        c = cb * _CC + i
        yh_c = seq_ref[c, :, _S:_T]
        ss_ref[i, 0] = yh_c

        lw = lw_ref[i]
        lw_max = jnp.max(lw, axis=1, keepdims=True)
        e = jnp.exp(lw - lw_max)
        w = e / jnp.sum(e, axis=1, keepdims=True)

        acc = w[0:1, 0:1] * yh_c
        for k in range(_K):
            ch = leaders_ref[c, k]
            st = starts_ref[c, k]
            d = deltas_ref[c, k]
            base = pl.multiple_of((st // 128) * 128, 128)
            off = st - base
            window = seq_ref[ch, :, pl.ds(base, _W)]
            gc = pltpu.roll(window, _W - off, axis=1)[:, :_P]
            gf = pltpu.roll(window, _W - (off + d), axis=1)[:, :_P]
            out_k = gf * wf_ref[c, k] + gc * wc_ref[c, k]
            ss_ref[i, 1 + k] = out_k
            acc = acc + w[0:1, 1 + k:2 + k] * out_k

        mean_c = mean_ref[i]
        std_c = std_ref[i]
        y_ref[i] = acc * std_c + mean_c


@jax.jit
def kernel(x, leaders, shifts, W, leader_weight):
    sh = jax.nn.sigmoid(shifts) * _S
    sf = jnp.floor(sh)
    sc = jnp.ceil(sh)
    padr = ((0, _CG - _C), (0, 0))
    starts = jnp.pad((_S - sc.astype(jnp.int32)).reshape(_C, _K), padr)
    deltas = jnp.pad((sc - sf).astype(jnp.int32).reshape(_C, _K), padr)
    wf = jnp.pad((sh - sf).reshape(_C, _K), padr)
    wc = jnp.pad((sh + 1.0 - sc).reshape(_C, _K), padr)
    leaders2 = jnp.pad(leaders.reshape(_C, _K), padr)

    x_t = jnp.pad(x.transpose(2, 0, 1), ((0, _CP - _C), (0, 0), (0, 0)))

    grid_a = (_CP // _CBLK,)
    seq_cbt, mean_cb, std_cb = pl.pallas_call(
        _stage_a_body,
        grid=grid_a,
        in_specs=[
            pl.BlockSpec((_CBLK, _B, _S), lambda i: (i, 0, 0)),
            pl.BlockSpec((_P, _S), lambda i: (0, 0)),
        ],
        out_specs=[
            pl.BlockSpec((_CBLK, _B, _TP), lambda i: (i, 0, 0)),
            pl.BlockSpec((_CBLK, _B), lambda i: (i, 0)),
            pl.BlockSpec((_CBLK, _B), lambda i: (i, 0)),
        ],
        out_shape=[
            jax.ShapeDtypeStruct((_CP, _B, _TP), jnp.float32),
            jax.ShapeDtypeStruct((_CP, _B), jnp.float32),
            jax.ShapeDtypeStruct((_CP, _B), jnp.float32),
        ],
    )(x_t, W)

    mean3 = mean_cb.reshape(_CP, _B, 1)
    std3 = std_cb.reshape(_CP, _B, 1)
    lw3 = jnp.pad(leader_weight, ((0, _CG - _C), (0, 0))).reshape(_CG, 1, 1 + _K)

    smem = functools.partial(pl.BlockSpec, memory_space=pltpu.SMEM)
    y_t, ss = pl.pallas_call(
        _stage_b_body,
        grid=(_CG // _CC,),
        in_specs=[
            pl.BlockSpec(memory_space=pltpu.MemorySpace.HBM),
            smem(),
            smem(),
            smem(),
            smem(),
            smem(),
            pl.BlockSpec((_CC, 1, 1 + _K), lambda c: (c, 0, 0)),
            pl.BlockSpec((_CC, _B, 1), lambda c: (c, 0, 0)),
            pl.BlockSpec((_CC, _B, 1), lambda c: (c, 0, 0)),
        ],
        out_specs=[
            pl.BlockSpec((_CC, _B, _P), lambda c: (c, 0, 0)),
            pl.BlockSpec((_CC, 1 + _K, _B, _P), lambda c: (c, 0, 0, 0)),
        ],
        out_shape=[
            jax.ShapeDtypeStruct((_C, _B, _P), jnp.float32),
            jax.ShapeDtypeStruct((_C, 1 + _K, _B, _P), jnp.float32),
        ],
        scratch_shapes=[
            pltpu.VMEM((_CP, _B, _TP), jnp.float32),
            pltpu.SemaphoreType.DMA,
        ],
    )(seq_cbt, leaders2, starts, deltas, wf, wc, lw3, mean3, std3)

    y = y_t.transpose(1, 2, 0)
    seq_shifted = ss.transpose(2, 3, 0, 1)
    return (y, seq_shifted)


# speedup vs baseline: 1.4017x; 1.1723x over previous
import functools

import jax
import jax.numpy as jnp
from jax.experimental import pallas as pl
from jax.experimental.pallas import tpu as pltpu

_B, _S, _P, _C, _K = 16, 720, 336, 321, 8
_T = _S + _P
_TP = 1152
_W = 512
_CP = 384
_CBLK = 128
_CC = 48
_CG = 336
_EPS = 1e-5


def _stage_a_body(xt_ref, w_ref, seq_ref, mean_ref, std_ref):
    xb = xt_ref[...]
    mean = jnp.mean(xb, axis=2)
    xc = xb - mean[:, :, None]
    var = jnp.mean(xc * xc, axis=2)
    std = jnp.sqrt(var + _EPS)
    xn = xc / std[:, :, None]
    mean_ref[...] = mean
    std_ref[...] = std
    seq_ref[:, :, :_S] = xn
    xn2 = xn.reshape(_CBLK * _B, _S)
    yh = jax.lax.dot_general(xn2, w_ref[...],
                             (((1,), (1,)), ((), ())),
                             preferred_element_type=jnp.float32)
    seq_ref[:, :, _S:_T] = yh.reshape(_CBLK, _B, _P)
    seq_ref[:, :, _T:] = jnp.zeros((_CBLK, _B, _TP - _T), jnp.float32)


def _stage_b_body(seq_hbm, leaders_ref, starts_ref, deltas_ref, wf_ref,
                  wc_ref, lw_ref, mean_ref, std_ref, y_ref, ss_ref,
                  seq_ref, sem):
    cb = pl.program_id(0)

    @pl.when(cb == 0)
    def _():
        cp = pltpu.make_async_copy(seq_hbm, seq_ref, sem)
        cp.start()
        cp.wait()

    for i in range(_CC):
        c = cb * _CC + i
        yh_c = seq_ref[c, :, _S:_T]
        ss_ref[i, 0] = yh_c

        lw = lw_ref[i]
        lw_max = jnp.max(lw, axis=1, keepdims=True)
        e = jnp.exp(lw - lw_max)
        w = e / jnp.sum(e, axis=1, keepdims=True)

        acc = w[0:1, 0:1] * yh_c
        for k in range(_K):
            ch = leaders_ref[c, k]
            st = starts_ref[c, k]
            d = deltas_ref[c, k]
            base = pl.multiple_of((st // 128) * 128, 128)
            off = st - base
            window = seq_ref[ch, :, pl.ds(base, _W)]
            gc = pltpu.roll(window, _W - off, axis=1)[:, :_P]
            gf = pltpu.roll(window, _W - (off + d), axis=1)[:, :_P]
            out_k = gf * wf_ref[c, k] + gc * wc_ref[c, k]
            ss_ref[i, 1 + k] = out_k
            acc = acc + w[0:1, 1 + k:2 + k] * out_k

        mean_c = mean_ref[i]
        std_c = std_ref[i]
        y_ref[i] = acc * std_c + mean_c


@jax.jit
def kernel(x, leaders, shifts, W, leader_weight):
    sh = jax.nn.sigmoid(shifts) * _S
    sf = jnp.floor(sh)
    sc = jnp.ceil(sh)
    padr = ((0, _CG - _C), (0, 0))
    starts = jnp.pad((_S - sc.astype(jnp.int32)).reshape(_C, _K), padr)
    deltas = jnp.pad((sc - sf).astype(jnp.int32).reshape(_C, _K), padr)
    wf = jnp.pad((sh - sf).reshape(_C, _K), padr)
    wc = jnp.pad((sh + 1.0 - sc).reshape(_C, _K), padr)
    leaders2 = jnp.pad(leaders.reshape(_C, _K), padr)

    x_t = jnp.pad(x.transpose(2, 0, 1), ((0, _CP - _C), (0, 0), (0, 0)))

    grid_a = (_CP // _CBLK,)
    seq_cbt, mean_cb, std_cb = pl.pallas_call(
        _stage_a_body,
        grid=grid_a,
        in_specs=[
            pl.BlockSpec((_CBLK, _B, _S), lambda i: (i, 0, 0)),
            pl.BlockSpec((_P, _S), lambda i: (0, 0)),
        ],
        out_specs=[
            pl.BlockSpec((_CBLK, _B, _TP), lambda i: (i, 0, 0)),
            pl.BlockSpec((_CBLK, _B), lambda i: (i, 0)),
            pl.BlockSpec((_CBLK, _B), lambda i: (i, 0)),
        ],
        out_shape=[
            jax.ShapeDtypeStruct((_CP, _B, _TP), jnp.float32),
            jax.ShapeDtypeStruct((_CP, _B), jnp.float32),
            jax.ShapeDtypeStruct((_CP, _B), jnp.float32),
        ],
    )(x_t, W)

    mean3 = mean_cb.reshape(_CP, _B, 1)
    std3 = std_cb.reshape(_CP, _B, 1)
    lw3 = jnp.pad(leader_weight, ((0, _CG - _C), (0, 0))).reshape(_CG, 1, 1 + _K)

    smem = functools.partial(pl.BlockSpec, memory_space=pltpu.SMEM)
    y_t, ss = pl.pallas_call(
        _stage_b_body,
        grid=(_CG // _CC,),
        in_specs=[
            pl.BlockSpec(memory_space=pltpu.MemorySpace.HBM),
            smem(),
            smem(),
            smem(),
            smem(),
            smem(),
            pl.BlockSpec((_CC, 1, 1 + _K), lambda c: (c, 0, 0)),
            pl.BlockSpec((_CC, _B, 1), lambda c: (c, 0, 0)),
            pl.BlockSpec((_CC, _B, 1), lambda c: (c, 0, 0)),
        ],
        out_specs=[
            pl.BlockSpec((_CC, _B, _P), lambda c: (c, 0, 0)),
            pl.BlockSpec((_CC, 1 + _K, _B, _P), lambda c: (c, 0, 0, 0)),
        ],
        out_shape=[
            jax.ShapeDtypeStruct((_C, _B, _P), jnp.float32),
            jax.ShapeDtypeStruct((_C, 1 + _K, _B, _P), jnp.float32),
        ],
        scratch_shapes=[
            pltpu.VMEM((_CP, _B, _TP), jnp.float32),
            pltpu.SemaphoreType.DMA,
        ],
    )(seq_cbt, leaders2, starts, deltas, wf, wc, lw3, mean3, std3)

    y = y_t.transpose(1, 2, 0)
    seq_shifted = ss.transpose(2, 3, 0, 1)
    return (y, seq_shifted)
